# baseline (speedup 1.0000x reference)
import jax
import jax.numpy as jnp
from jax.experimental import pallas as pl
from jax.experimental.pallas import tpu as pltpu
from jax.experimental.pallas import tpu_sc as plsc

N = 10000
K = 16
C_IN = 128
C_OUT = 128
S = 8
MID = 128
HID = MID // S
EPS = 1e-5

NP = 10240
NPK = NP * K
QB = 128
CB = 1024
NCHUNK = NP // CB
EB = QB * K
NBLK = NP // QB
NE = N * K
IMAX = 2**31 - 1
HUGE = 3e18


def _bf16_dot(a, b):
    return jax.lax.dot(a.astype(jnp.bfloat16), b.astype(jnp.bfloat16),
                       preferred_element_type=jnp.float32)


def _knn_kernel(a_ref, b_ref, sq_ref, dist_ref, idx_ref, kscra, kscrb, candk, candi):
    b_blk = b_ref[...]
    sqq = jnp.broadcast_to(sq_ref[0:1, :], (CB, QB))
    sub16 = jax.lax.broadcasted_iota(jnp.int32, (K, QB), 0)

    def keys_for(c):
        a_chunk = a_ref[pl.ds(c * CB, CB), :]
        sqc = jnp.broadcast_to(a_chunk[:, 3:4], (CB, QB))
        dotv = jax.lax.dot(a_chunk.astype(jnp.bfloat16), b_blk,
                           preferred_element_type=jnp.float32)
        d2 = (sqc + sqq) - 2.0 * dotv
        d2 = jnp.maximum(d2, 0.0)
        return jax.lax.bitcast_convert_type(d2, jnp.int32)

    def step(kk, scr, riota, i, bk, bi, base):
        m = jnp.min(kk, axis=0, keepdims=True)
        sel = jnp.where(kk == m, riota, IMAX)
        r = jnp.min(sel, axis=0, keepdims=True)
        scr[...] = jnp.where(riota == r, IMAX, kk)
        onrow = sub16 == i
        bk = jnp.where(onrow, jnp.broadcast_to(m, (K, QB)), bk)
        bi = jnp.where(onrow, jnp.broadcast_to(r + base, (K, QB)), bi)
        return bk, bi

    riota = jax.lax.broadcasted_iota(jnp.int32, (CB, QB), 0)
    for c in range(0, NCHUNK, 2):
        kscra[...] = keys_for(c)
        kscrb[...] = keys_for(c + 1)

        def it(i, carry):
            bka, bia, bkb, bib = carry
            bka, bia = step(kscra[...], kscra, riota, i, bka, bia, jnp.int32(c * CB))
            bkb, bib = step(kscrb[...], kscrb, riota, i, bkb, bib, jnp.int32((c + 1) * CB))
            return bka, bia, bkb, bib

        z16 = jnp.zeros((K, QB), jnp.int32)
        bka, bia, bkb, bib = jax.lax.fori_loop(
            0, K, it, (jnp.full((K, QB), IMAX, jnp.int32), z16,
                       jnp.full((K, QB), IMAX, jnp.int32), z16))
        candk[pl.ds(c * K, K), :] = bka
        candi[pl.ds(c * K, K), :] = bia
        candk[pl.ds((c + 1) * K, K), :] = bkb
        candi[pl.ds((c + 1) * K, K), :] = bib

    riota_m = jax.lax.broadcasted_iota(jnp.int32, (NCHUNK * K, QB), 0)

    def itm(i, carry):
        bk, bi = carry
        kk = candk[...]
        m = jnp.min(kk, axis=0, keepdims=True)
        sel = jnp.where(kk == m, riota_m, IMAX)
        r = jnp.min(sel, axis=0, keepdims=True)
        gi = jnp.min(jnp.where(riota_m == r, candi[...], IMAX), axis=0, keepdims=True)
        candk[...] = jnp.where(riota_m == r, IMAX, kk)
        onrow = sub16 == i
        bk = jnp.where(onrow, jnp.broadcast_to(m, (K, QB)), bk)
        bi = jnp.where(onrow, jnp.broadcast_to(gi, (K, QB)), bi)
        return bk, bi

    bk, bi = jax.lax.fori_loop(
        0, K, itm, (jnp.full((K, QB), IMAX, jnp.int32), jnp.zeros((K, QB), jnp.int32)))
    dist_ref[...] = jnp.sqrt(jax.lax.bitcast_convert_type(bk, jnp.float32))
    idx_ref[...] = bi


def _knn(p):
    sq = jnp.sum(p * p, axis=1)
    zeros = jnp.zeros((N,), jnp.float32)
    a = jnp.stack([p[:, 0], p[:, 1], p[:, 2], sq] + [zeros] * 12, axis=1)
    a_pad = jnp.zeros((NP - N, 16), jnp.float32).at[:, 3].set(HUGE)
    a = jnp.concatenate([a, a_pad], axis=0)
    b = jnp.stack([p[:, 0], p[:, 1], p[:, 2]] + [zeros] * 13, axis=0)
    b = jnp.concatenate([b, jnp.zeros((16, NP - N), jnp.float32)], axis=1)
    b = b.astype(jnp.bfloat16)
    sqq = jnp.concatenate([sq, jnp.zeros((NP - N,), jnp.float32)])
    sqq = jnp.broadcast_to(sqq[None, :], (8, NP))

    dist16, idx16 = pl.pallas_call(
        _knn_kernel,
        grid=(NBLK,),
        in_specs=[
            pl.BlockSpec((NP, 16), lambda i: (0, 0)),
            pl.BlockSpec((16, QB), lambda i: (0, i)),
            pl.BlockSpec((8, QB), lambda i: (0, i)),
        ],
        out_specs=[
            pl.BlockSpec((K, QB), lambda i: (0, i)),
            pl.BlockSpec((K, QB), lambda i: (0, i)),
        ],
        out_shape=[
            jax.ShapeDtypeStruct((K, NP), jnp.float32),
            jax.ShapeDtypeStruct((K, NP), jnp.int32),
        ],
        scratch_shapes=[
            pltpu.VMEM((CB, QB), jnp.int32),
            pltpu.VMEM((CB, QB), jnp.int32),
            pltpu.VMEM((NCHUNK * K, QB), jnp.int32),
            pltpu.VMEM((NCHUNK * K, QB), jnp.int32),
        ],
    )(a, b, sqq)
    return dist16, idx16


def _sc_gather(table, idx_e):
    mesh = plsc.VectorSubcoreMesh(core_axis_name="core", subcore_axis_name="subcore")

    @pl.kernel(out_type=jax.ShapeDtypeStruct((NPK, 128), jnp.int32),
               mesh=mesh)
    def gk(t_hbm, i_hbm, o_hbm):
        def body(i_vmem, o_vmem):
            pltpu.sync_copy(t_hbm.at[i_vmem.at[0]], o_vmem)

        pltpu.emit_pipeline(
            body,
            grid=(NPK // 128,),
            in_specs=[pl.BlockSpec((1, 128), lambda i: (0, i))],
            out_specs=[pl.BlockSpec((128, 128), lambda i: (i, 0))],
            core_axis_name=("core", "subcore"),
            dimension_semantics=(pltpu.PARALLEL,),
        )(i_hbm, o_hbm)

    return gk(table, idx_e)


def _rep16(a):
    return jnp.broadcast_to(a[:, None, :], (QB, K, a.shape[1])).reshape(EB, a.shape[1])


def _valid_mask(i):
    row = jax.lax.broadcasted_iota(jnp.int32, (EB, 1), 0) + i * EB
    return row < NE


def _accum_stats(i, stat_ref, s1, s2):
    pad = jnp.zeros((6, 128), jnp.float32)
    upd = jnp.concatenate([s1, s2, pad], axis=0)

    @pl.when(i == 0)
    def _():
        stat_ref[...] = jnp.zeros((8, 128), jnp.float32)

    stat_ref[...] += upd


def _pad128(v):
    return jnp.concatenate([v, jnp.zeros((1, 128 - v.shape[1]), jnp.float32)], axis=1)


def _p1_kernel(gg_ref, gc_ref, we1_ref, be1_ref, h_ref, stat_ref):
    i = pl.program_id(0)
    gg = jax.lax.bitcast_convert_type(gg_ref[:, 64:70], jnp.float32)
    gc = _rep16(gc_ref[...])
    pe = gg[:, 0:3] - gc[:, 0:3]
    dr = jnp.abs(pe[:, 0:1])
    dtheta = jnp.abs(jnp.remainder(pe[:, 1:2] + jnp.pi, 2.0 * jnp.pi) - jnp.pi)
    dz = jnp.abs(pe[:, 2:3])
    ne = gg[:, 3:6] - gc[:, 3:6]
    dn = jnp.sqrt(jnp.sum(ne * ne, axis=1, keepdims=True))
    ef = jnp.concatenate([dn, dtheta, dz, dr] + [jnp.zeros((EB, 12), jnp.float32)],
                         axis=1)
    h = _bf16_dot(ef, we1_ref[...]) + be1_ref[...]
    h_ref[...] = h
    hm = jnp.where(_valid_mask(i), h, 0.0)
    s1 = jnp.sum(hm, axis=0, keepdims=True)
    s2 = jnp.sum(hm * hm, axis=0, keepdims=True)
    _accum_stats(i, stat_ref, _pad128(s1), _pad128(s2))


def _p2_kernel(xg_ref, h_ref, expd_ref, xc_ref, bn1_ref,
               wq_ref, bq_ref, wk_ref, bk_ref, wv_ref, bv_ref,
               we2_ref, be2_ref, wpre_ref, av_ref, stat_ref):
    i = pl.program_id(0)
    xc = xc_ref[...]
    xq_c = _bf16_dot(xc, wq_ref[...]) + bq_ref[...]
    xe = xg_ref[...].astype(jnp.float32) - _rep16(xc)
    xk = _bf16_dot(xe, wk_ref[...]) + bk_ref[...]
    xv = _bf16_dot(xe, wv_ref[...]) + bv_ref[...]
    h = h_ref[...] * bn1_ref[0:1, 0:16] + bn1_ref[1:2, 0:16]
    h = jnp.maximum(h, 0.0)
    emb = _bf16_dot(h, we2_ref[...]) + be2_ref[...]
    wpre = (_rep16(xq_c) - xk) + emb
    wpre_ref[...] = wpre
    expd = jnp.broadcast_to(expd_ref[...], (EB, C_OUT))
    av_ref[...] = xv * expd + emb
    wm = jnp.where(_valid_mask(i), wpre, 0.0)
    s1 = jnp.sum(wm, axis=0, keepdims=True)
    s2 = jnp.sum(wm * wm, axis=0, keepdims=True)
    _accum_stats(i, stat_ref, s1, s2)


def _p3_kernel(wpre_ref, bn2_ref, ww1_ref, bw1_ref, w1_ref, stat_ref):
    i = pl.program_id(0)
    wb = wpre_ref[...] * bn2_ref[0:1, :] + bn2_ref[1:2, :]
    wb = jnp.maximum(wb, 0.0)
    w1 = _bf16_dot(wb, ww1_ref[...]) + bw1_ref[...]
    w1_ref[...] = w1
    wm = jnp.where(_valid_mask(i), w1, 0.0)
    s1 = jnp.sum(wm, axis=0, keepdims=True)
    s2 = jnp.sum(wm * wm, axis=0, keepdims=True)
    _accum_stats(i, stat_ref, _pad128(s1), _pad128(s2))


def _p4_kernel(w1_ref, av_ref, bn3_ref, ww2_ref, bw2_ref, out_ref):
    w1 = w1_ref[...] * bn3_ref[0:1, 0:16] + bn3_ref[1:2, 0:16]
    w1 = jnp.maximum(w1, 0.0)
    w2 = _bf16_dot(w1, ww2_ref[...]) + bw2_ref[...]
    w3 = w2.reshape(QB, K, HID)
    m = jnp.max(w3, axis=1, keepdims=True)
    e = jnp.exp(w3 - m)
    w = e / jnp.sum(e, axis=1, keepdims=True)
    wt = jnp.concatenate([w] * S, axis=2)
    av = av_ref[...].reshape(QB, K, C_OUT)
    out_ref[...] = jnp.sum(av * wt, axis=1)


def _bn_params(stat, denom, g, b, nch):
    s1 = stat[0, :nch]
    s2 = stat[1, :nch]
    mean = s1 / denom
    var = s2 / denom - mean * mean
    scale = g / jnp.sqrt(var + EPS)
    shift = b - mean * scale
    lanes = jnp.zeros((2, 128), jnp.float32)
    lanes = lanes.at[0, :nch].set(scale).at[1, :nch].set(shift)
    return jnp.concatenate([lanes, jnp.zeros((6, 128), jnp.float32)], axis=0)


def _edge_grid_call(kernel_fn, in_arrs, in_specs, out_specs, out_shape):
    return pl.pallas_call(
        kernel_fn,
        grid=(NBLK,),
        in_specs=in_specs,
        out_specs=out_specs,
        out_shape=out_shape,
    )(*in_arrs)


def kernel(p, n, x, o, Wq, bq, Wk, bk, Wv, bv, We1, be1, g_e, b_e, We2, be2, g_w0, b_w0, Ww1, bw1, g_w1, b_w1, Ww2, bw2):
    f32 = jnp.float32
    dist16, idx16 = _knn(p)
    idx = idx16[:, :N].T
    dist = dist16[:, :N].T

    idx_e = jnp.concatenate([idx.reshape(-1), jnp.zeros((NPK - NE,), jnp.int32)])
    idx_e = idx_e[None, :]
    expd = jnp.exp(-dist).reshape(-1)
    expd_e = jnp.concatenate([expd, jnp.zeros((NPK - NE,), f32)])[:, None]

    x_pad = jnp.concatenate([x, jnp.zeros((NP - N, C_IN), f32)], axis=0)
    r = jnp.sqrt(p[:, 0] ** 2 + p[:, 1] ** 2)
    theta = jnp.arctan2(p[:, 1], p[:, 0])
    geo6 = jnp.stack([r, theta, p[:, 2], n[:, 0], n[:, 1], n[:, 2]], axis=1)
    geo6 = jnp.concatenate([geo6, jnp.zeros((NP - N, 6), f32)], axis=0)
    geo = jnp.concatenate([geo6, jnp.zeros((NP, 10), f32)], axis=1)
    x_packed = jax.lax.bitcast_convert_type(
        x_pad.astype(jnp.bfloat16).reshape(NP, 64, 2), jnp.int32)
    geo_bits = jax.lax.bitcast_convert_type(geo6, jnp.int32)
    table = jnp.concatenate(
        [x_packed, geo_bits, jnp.zeros((NP, 58), jnp.int32)], axis=1)

    tg = _sc_gather(table, idx_e)
    xg_bf = jax.lax.bitcast_convert_type(
        tg[:, :64], jnp.bfloat16).reshape(NPK, 128)

    full = lambda shp: pl.BlockSpec(shp, lambda i: tuple(0 for _ in shp))
    eb_blk = lambda L: pl.BlockSpec((EB, L), lambda i: (i, 0))
    qb_blk = lambda L: pl.BlockSpec((QB, L), lambda i: (i, 0))
    acc_spec = pl.BlockSpec((8, 128), lambda i: (0, 0))

    We1_16 = jnp.concatenate([We1, jnp.zeros((12, 16), f32)], axis=0)
    h_pre, st1 = _edge_grid_call(
        _p1_kernel,
        [tg, geo, We1_16, be1[None, :]],
        [pl.BlockSpec((EB, 128), lambda i: (i, 0)), qb_blk(16), full((16, 16)), full((1, 16))],
        [pl.BlockSpec((EB, 16), lambda i: (i, 0)), acc_spec],
        [jax.ShapeDtypeStruct((NPK, 16), f32), jax.ShapeDtypeStruct((8, 128), f32)],
    )
    bn1 = _bn_params(st1, float(NE), g_e, b_e, 16)

    w_pre, a_v, st2 = _edge_grid_call(
        _p2_kernel,
        [xg_bf, h_pre, expd_e, x_pad, bn1,
         Wq, bq[None, :], Wk, bk[None, :], Wv, bv[None, :], We2, be2[None, :]],
        [pl.BlockSpec((EB, C_IN), lambda i: (i, 0)), pl.BlockSpec((EB, 16), lambda i: (i, 0)),
         pl.BlockSpec((EB, 1), lambda i: (i, 0)), qb_blk(C_IN), full((8, 128)),
         full((C_IN, MID)), full((1, MID)), full((C_IN, MID)), full((1, MID)),
         full((C_IN, C_OUT)), full((1, C_OUT)), full((16, C_OUT)), full((1, C_OUT))],
        [eb_blk(MID), eb_blk(C_OUT), acc_spec],
        [jax.ShapeDtypeStruct((NPK, MID), f32), jax.ShapeDtypeStruct((NPK, C_OUT), f32),
         jax.ShapeDtypeStruct((8, 128), f32)],
    )
    bn2 = _bn_params(st2, float(NE), g_w0, b_w0, 128)

    w1_pre, st3 = _edge_grid_call(
        _p3_kernel,
        [w_pre, bn2, Ww1, bw1[None, :]],
        [eb_blk(MID), full((8, 128)), full((MID, HID)), full((1, HID))],
        [pl.BlockSpec((EB, HID), lambda i: (i, 0)), acc_spec],
        [jax.ShapeDtypeStruct((NPK, HID), f32), jax.ShapeDtypeStruct((8, 128), f32)],
    )
    bn3 = _bn_params(st3, float(NE), g_w1, b_w1, HID)

    out = _edge_grid_call(
        _p4_kernel,
        [w1_pre, a_v, bn3, Ww2, bw2[None, :]],
        [pl.BlockSpec((EB, HID), lambda i: (i, 0)), eb_blk(C_OUT), full((8, 128)),
         full((HID, HID)), full((1, HID))],
        qb_blk(C_OUT),
        jax.ShapeDtypeStruct((NP, C_OUT), f32),
    )
    return out[:N]


# speedup vs baseline: 1.0797x; 1.0797x over previous
import jax
import jax.numpy as jnp
from jax.experimental import pallas as pl
from jax.experimental.pallas import tpu as pltpu
from jax.experimental.pallas import tpu_sc as plsc

N = 10000
K = 16
C_IN = 128
C_OUT = 128
S = 8
MID = 128
HID = MID // S
EPS = 1e-5

NP = 10240
NPK = NP * K
QB = 128
CB = 1024
NCHUNK = NP // CB
EB = QB * K
NBLK = NP // QB
NE = N * K
IMAX = 2**31 - 1
HUGE = 3e18


def _bf16_dot(a, b):
    return jax.lax.dot(a.astype(jnp.bfloat16), b.astype(jnp.bfloat16),
                       preferred_element_type=jnp.float32)


def _knn_kernel(a_ref, b_ref, sq_ref, dist_ref, idx_ref, kscra, kscrb, candk, candi):
    b_blk = b_ref[...]
    sqq = jnp.broadcast_to(sq_ref[0:1, :], (CB, QB))
    sub16 = jax.lax.broadcasted_iota(jnp.int32, (K, QB), 0)

    def keys_for(c):
        a_chunk = a_ref[pl.ds(c * CB, CB), :]
        sqc = jnp.broadcast_to(a_chunk[:, 3:4], (CB, QB))
        dotv = jax.lax.dot(a_chunk.astype(jnp.bfloat16), b_blk,
                           preferred_element_type=jnp.float32)
        d2 = (sqc + sqq) - 2.0 * dotv
        d2 = jnp.maximum(d2, 0.0)
        return jax.lax.bitcast_convert_type(d2, jnp.int32)

    def step(kk, scr, riota, i, bk, bi, base):
        m = jnp.min(kk, axis=0, keepdims=True)
        sel = jnp.where(kk == m, riota, IMAX)
        r = jnp.min(sel, axis=0, keepdims=True)
        scr[...] = jnp.where(riota == r, IMAX, kk)
        onrow = sub16 == i
        bk = jnp.where(onrow, jnp.broadcast_to(m, (K, QB)), bk)
        bi = jnp.where(onrow, jnp.broadcast_to(r + base, (K, QB)), bi)
        return bk, bi

    riota = jax.lax.broadcasted_iota(jnp.int32, (CB, QB), 0)
    for c in range(0, NCHUNK, 2):
        kscra[...] = keys_for(c)
        kscrb[...] = keys_for(c + 1)

        def it(i, carry):
            bka, bia, bkb, bib = carry
            bka, bia = step(kscra[...], kscra, riota, i, bka, bia, jnp.int32(c * CB))
            bkb, bib = step(kscrb[...], kscrb, riota, i, bkb, bib, jnp.int32((c + 1) * CB))
            return bka, bia, bkb, bib

        z16 = jnp.zeros((K, QB), jnp.int32)
        bka, bia, bkb, bib = jax.lax.fori_loop(
            0, K, it, (jnp.full((K, QB), IMAX, jnp.int32), z16,
                       jnp.full((K, QB), IMAX, jnp.int32), z16))
        candk[pl.ds(c * K, K), :] = bka
        candi[pl.ds(c * K, K), :] = bia
        candk[pl.ds((c + 1) * K, K), :] = bkb
        candi[pl.ds((c + 1) * K, K), :] = bib

    riota_m = jax.lax.broadcasted_iota(jnp.int32, (NCHUNK * K, QB), 0)

    def itm(i, carry):
        bk, bi = carry
        kk = candk[...]
        m = jnp.min(kk, axis=0, keepdims=True)
        sel = jnp.where(kk == m, riota_m, IMAX)
        r = jnp.min(sel, axis=0, keepdims=True)
        gi = jnp.min(jnp.where(riota_m == r, candi[...], IMAX), axis=0, keepdims=True)
        candk[...] = jnp.where(riota_m == r, IMAX, kk)
        onrow = sub16 == i
        bk = jnp.where(onrow, jnp.broadcast_to(m, (K, QB)), bk)
        bi = jnp.where(onrow, jnp.broadcast_to(gi, (K, QB)), bi)
        return bk, bi

    bk, bi = jax.lax.fori_loop(
        0, K, itm, (jnp.full((K, QB), IMAX, jnp.int32), jnp.zeros((K, QB), jnp.int32)))
    dist_ref[...] = jnp.sqrt(jax.lax.bitcast_convert_type(bk, jnp.float32))
    idx_ref[...] = bi


def _knn(p):
    sq = jnp.sum(p * p, axis=1)
    zeros = jnp.zeros((N,), jnp.float32)
    a = jnp.stack([p[:, 0], p[:, 1], p[:, 2], sq] + [zeros] * 12, axis=1)
    a_pad = jnp.zeros((NP - N, 16), jnp.float32).at[:, 3].set(HUGE)
    a = jnp.concatenate([a, a_pad], axis=0)
    b = jnp.stack([p[:, 0], p[:, 1], p[:, 2]] + [zeros] * 13, axis=0)
    b = jnp.concatenate([b, jnp.zeros((16, NP - N), jnp.float32)], axis=1)
    b = b.astype(jnp.bfloat16)
    sqq = jnp.concatenate([sq, jnp.zeros((NP - N,), jnp.float32)])
    sqq = jnp.broadcast_to(sqq[None, :], (8, NP))

    dist16, idx16 = pl.pallas_call(
        _knn_kernel,
        grid=(NBLK,),
        in_specs=[
            pl.BlockSpec((NP, 16), lambda i: (0, 0)),
            pl.BlockSpec((16, QB), lambda i: (0, i)),
            pl.BlockSpec((8, QB), lambda i: (0, i)),
        ],
        out_specs=[
            pl.BlockSpec((K, QB), lambda i: (0, i)),
            pl.BlockSpec((K, QB), lambda i: (0, i)),
        ],
        out_shape=[
            jax.ShapeDtypeStruct((K, NP), jnp.float32),
            jax.ShapeDtypeStruct((K, NP), jnp.int32),
        ],
        scratch_shapes=[
            pltpu.VMEM((CB, QB), jnp.int32),
            pltpu.VMEM((CB, QB), jnp.int32),
            pltpu.VMEM((NCHUNK * K, QB), jnp.int32),
            pltpu.VMEM((NCHUNK * K, QB), jnp.int32),
        ],
    )(a, b, sqq)
    return dist16, idx16


def _sc_gather(table, idx_e):
    mesh = plsc.VectorSubcoreMesh(core_axis_name="core", subcore_axis_name="subcore")

    @pl.kernel(out_type=jax.ShapeDtypeStruct((NPK, 256), jnp.float32),
               mesh=mesh)
    def gk(t_hbm, i_hbm, o_hbm):
        def body(i_vmem, o_vmem):
            pltpu.sync_copy(t_hbm.at[i_vmem.at[0]], o_vmem)

        pltpu.emit_pipeline(
            body,
            grid=(NPK // 128,),
            in_specs=[pl.BlockSpec((1, 128), lambda i: (0, i))],
            out_specs=[pl.BlockSpec((128, 256), lambda i: (i, 0))],
            core_axis_name=("core", "subcore"),
            dimension_semantics=(pltpu.PARALLEL,),
        )(i_hbm, o_hbm)

    return gk(table, idx_e)


def _rep16(a):
    return jnp.broadcast_to(a[:, None, :], (QB, K, a.shape[1])).reshape(EB, a.shape[1])


def _valid_mask(i):
    row = jax.lax.broadcasted_iota(jnp.int32, (EB, 1), 0) + i * EB
    return row < NE


def _accum_stats(i, stat_ref, s1, s2):
    pad = jnp.zeros((6, 128), jnp.float32)
    upd = jnp.concatenate([s1, s2, pad], axis=0)

    @pl.when(i == 0)
    def _():
        stat_ref[...] = jnp.zeros((8, 128), jnp.float32)

    stat_ref[...] += upd


def _pad128(v):
    return jnp.concatenate([v, jnp.zeros((1, 128 - v.shape[1]), jnp.float32)], axis=1)


def _p1_kernel(gg_ref, gc_ref, we1_ref, be1_ref, h_ref, stat_ref):
    i = pl.program_id(0)
    gg = gg_ref[...]
    gc = _rep16(gc_ref[...])
    pe = gg[:, 0:3] - gc[:, 0:3]
    dr = jnp.abs(pe[:, 0:1])
    dtheta = jnp.abs(jnp.remainder(pe[:, 1:2] + jnp.pi, 2.0 * jnp.pi) - jnp.pi)
    dz = jnp.abs(pe[:, 2:3])
    ne = gg[:, 3:6] - gc[:, 3:6]
    dn = jnp.sqrt(jnp.sum(ne * ne, axis=1, keepdims=True))
    ef = jnp.concatenate([dn, dtheta, dz, dr] + [jnp.zeros((EB, 12), jnp.float32)],
                         axis=1)
    h = _bf16_dot(ef, we1_ref[...]) + be1_ref[...]
    h_ref[...] = h
    hm = jnp.where(_valid_mask(i), h, 0.0)
    s1 = jnp.sum(hm, axis=0, keepdims=True)
    s2 = jnp.sum(hm * hm, axis=0, keepdims=True)
    _accum_stats(i, stat_ref, _pad128(s1), _pad128(s2))


def _p2_kernel(xg_ref, h_ref, expd_ref, xc_ref, bn1_ref,
               wq_ref, bq_ref, wk_ref, bk_ref, wv_ref, bv_ref,
               we2_ref, be2_ref, wpre_ref, av_ref, stat_ref):
    i = pl.program_id(0)
    xc = xc_ref[...]
    xq_c = _bf16_dot(xc, wq_ref[...]) + bq_ref[...]
    xe = xg_ref[...].astype(jnp.float32) - _rep16(xc)
    xk = _bf16_dot(xe, wk_ref[...]) + bk_ref[...]
    xv = _bf16_dot(xe, wv_ref[...]) + bv_ref[...]
    h = h_ref[...] * bn1_ref[0:1, 0:16] + bn1_ref[1:2, 0:16]
    h = jnp.maximum(h, 0.0)
    emb = _bf16_dot(h, we2_ref[...]) + be2_ref[...]
    wpre = (_rep16(xq_c) - xk) + emb
    wpre_ref[...] = wpre
    expd = jnp.broadcast_to(expd_ref[...], (EB, C_OUT))
    av_ref[...] = xv * expd + emb
    wm = jnp.where(_valid_mask(i), wpre, 0.0)
    s1 = jnp.sum(wm, axis=0, keepdims=True)
    s2 = jnp.sum(wm * wm, axis=0, keepdims=True)
    _accum_stats(i, stat_ref, s1, s2)


def _p3_kernel(wpre_ref, bn2_ref, ww1_ref, bw1_ref, w1_ref, stat_ref):
    i = pl.program_id(0)
    wb = wpre_ref[...] * bn2_ref[0:1, :] + bn2_ref[1:2, :]
    wb = jnp.maximum(wb, 0.0)
    w1 = _bf16_dot(wb, ww1_ref[...]) + bw1_ref[...]
    w1_ref[...] = w1
    wm = jnp.where(_valid_mask(i), w1, 0.0)
    s1 = jnp.sum(wm, axis=0, keepdims=True)
    s2 = jnp.sum(wm * wm, axis=0, keepdims=True)
    _accum_stats(i, stat_ref, _pad128(s1), _pad128(s2))


def _p4_kernel(w1_ref, av_ref, bn3_ref, ww2_ref, bw2_ref, out_ref):
    w1 = w1_ref[...] * bn3_ref[0:1, 0:16] + bn3_ref[1:2, 0:16]
    w1 = jnp.maximum(w1, 0.0)
    w2 = _bf16_dot(w1, ww2_ref[...]) + bw2_ref[...]
    w3 = w2.reshape(QB, K, HID)
    m = jnp.max(w3, axis=1, keepdims=True)
    e = jnp.exp(w3 - m)
    w = e / jnp.sum(e, axis=1, keepdims=True)
    wt = jnp.concatenate([w] * S, axis=2)
    av = av_ref[...].reshape(QB, K, C_OUT)
    out_ref[...] = jnp.sum(av * wt, axis=1)


def _bn_params(stat, denom, g, b, nch):
    s1 = stat[0, :nch]
    s2 = stat[1, :nch]
    mean = s1 / denom
    var = s2 / denom - mean * mean
    scale = g / jnp.sqrt(var + EPS)
    shift = b - mean * scale
    lanes = jnp.zeros((2, 128), jnp.float32)
    lanes = lanes.at[0, :nch].set(scale).at[1, :nch].set(shift)
    return jnp.concatenate([lanes, jnp.zeros((6, 128), jnp.float32)], axis=0)


def _edge_grid_call(kernel_fn, in_arrs, in_specs, out_specs, out_shape):
    return pl.pallas_call(
        kernel_fn,
        grid=(NBLK,),
        in_specs=in_specs,
        out_specs=out_specs,
        out_shape=out_shape,
    )(*in_arrs)


def kernel(p, n, x, o, Wq, bq, Wk, bk, Wv, bv, We1, be1, g_e, b_e, We2, be2, g_w0, b_w0, Ww1, bw1, g_w1, b_w1, Ww2, bw2):
    f32 = jnp.float32
    dist16, idx16 = _knn(p)
    idx = idx16[:, :N].T
    dist = dist16[:, :N].T

    idx_e = jnp.concatenate([idx.reshape(-1), jnp.zeros((NPK - NE,), jnp.int32)])
    idx_e = idx_e[None, :]
    expd = jnp.exp(-dist).reshape(-1)
    expd_e = jnp.concatenate([expd, jnp.zeros((NPK - NE,), f32)])[:, None]

    x_pad = jnp.concatenate([x, jnp.zeros((NP - N, C_IN), f32)], axis=0)
    r = jnp.sqrt(p[:, 0] ** 2 + p[:, 1] ** 2)
    theta = jnp.arctan2(p[:, 1], p[:, 0])
    geo6 = jnp.stack([r, theta, p[:, 2], n[:, 0], n[:, 1], n[:, 2]], axis=1)
    geo6 = jnp.concatenate([geo6, jnp.zeros((NP - N, 6), f32)], axis=0)
    geo = jnp.concatenate([geo6, jnp.zeros((NP, 10), f32)], axis=1)
    table = jnp.concatenate([x_pad, geo6, jnp.zeros((NP, 122), f32)], axis=1)

    tg = _sc_gather(table, idx_e)

    full = lambda shp: pl.BlockSpec(shp, lambda i: tuple(0 for _ in shp))
    eb_blk = lambda L: pl.BlockSpec((EB, L), lambda i: (i, 0))
    qb_blk = lambda L: pl.BlockSpec((QB, L), lambda i: (i, 0))
    acc_spec = pl.BlockSpec((8, 128), lambda i: (0, 0))

    We1_16 = jnp.concatenate([We1, jnp.zeros((12, 16), f32)], axis=0)
    h_pre, st1 = _edge_grid_call(
        _p1_kernel,
        [tg, geo, We1_16, be1[None, :]],
        [pl.BlockSpec((EB, 128), lambda i: (i, 1)), qb_blk(16), full((16, 16)), full((1, 16))],
        [pl.BlockSpec((EB, 16), lambda i: (i, 0)), acc_spec],
        [jax.ShapeDtypeStruct((NPK, 16), f32), jax.ShapeDtypeStruct((8, 128), f32)],
    )
    bn1 = _bn_params(st1, float(NE), g_e, b_e, 16)

    w_pre, a_v, st2 = _edge_grid_call(
        _p2_kernel,
        [tg, h_pre, expd_e, x_pad, bn1,
         Wq, bq[None, :], Wk, bk[None, :], Wv, bv[None, :], We2, be2[None, :]],
        [pl.BlockSpec((EB, C_IN), lambda i: (i, 0)), pl.BlockSpec((EB, 16), lambda i: (i, 0)),
         pl.BlockSpec((EB, 1), lambda i: (i, 0)), qb_blk(C_IN), full((8, 128)),
         full((C_IN, MID)), full((1, MID)), full((C_IN, MID)), full((1, MID)),
         full((C_IN, C_OUT)), full((1, C_OUT)), full((16, C_OUT)), full((1, C_OUT))],
        [eb_blk(MID), eb_blk(C_OUT), acc_spec],
        [jax.ShapeDtypeStruct((NPK, MID), f32), jax.ShapeDtypeStruct((NPK, C_OUT), f32),
         jax.ShapeDtypeStruct((8, 128), f32)],
    )
    bn2 = _bn_params(st2, float(NE), g_w0, b_w0, 128)

    w1_pre, st3 = _edge_grid_call(
        _p3_kernel,
        [w_pre, bn2, Ww1, bw1[None, :]],
        [eb_blk(MID), full((8, 128)), full((MID, HID)), full((1, HID))],
        [pl.BlockSpec((EB, HID), lambda i: (i, 0)), acc_spec],
        [jax.ShapeDtypeStruct((NPK, HID), f32), jax.ShapeDtypeStruct((8, 128), f32)],
    )
    bn3 = _bn_params(st3, float(NE), g_w1, b_w1, HID)

    out = _edge_grid_call(
        _p4_kernel,
        [w1_pre, a_v, bn3, Ww2, bw2[None, :]],
        [pl.BlockSpec((EB, HID), lambda i: (i, 0)), eb_blk(C_OUT), full((8, 128)),
         full((HID, HID)), full((1, HID))],
        qb_blk(C_OUT),
        jax.ShapeDtypeStruct((NP, C_OUT), f32),
    )
    return out[:N]


# speedup vs baseline: 1.0953x; 1.0145x over previous
import jax
import jax.numpy as jnp
from jax.experimental import pallas as pl
from jax.experimental.pallas import tpu as pltpu
from jax.experimental.pallas import tpu_sc as plsc

N = 10000
K = 16
C_IN = 128
C_OUT = 128
S = 8
MID = 128
HID = MID // S
EPS = 1e-5

NP = 10240
NPK = NP * K
QB = 128
CB = 512
NCHUNK = NP // CB
ILP = 4
EB = QB * K
NBLK = NP // QB
NE = N * K
IMAX = 2**31 - 1
HUGE = 3e18


def _bf16_dot(a, b):
    return jax.lax.dot(a.astype(jnp.bfloat16), b.astype(jnp.bfloat16),
                       preferred_element_type=jnp.float32)


def _knn_kernel(a_ref, b_ref, sq_ref, dist_ref, idx_ref, kscra, kscrb, kscrc, kscrd, candk, candi):
    b_blk = b_ref[...]
    sqq = jnp.broadcast_to(sq_ref[0:1, :], (CB, QB))
    sub16 = jax.lax.broadcasted_iota(jnp.int32, (K, QB), 0)

    def keys_for(c):
        a_chunk = a_ref[pl.ds(c * CB, CB), :]
        sqc = jnp.broadcast_to(a_chunk[:, 3:4], (CB, QB))
        dotv = jax.lax.dot(a_chunk.astype(jnp.bfloat16), b_blk,
                           preferred_element_type=jnp.float32)
        d2 = (sqc + sqq) - 2.0 * dotv
        d2 = jnp.maximum(d2, 0.0)
        return jax.lax.bitcast_convert_type(d2, jnp.int32)

    def step(kk, scr, riota, i, bk, bi, base):
        m = jnp.min(kk, axis=0, keepdims=True)
        sel = jnp.where(kk == m, riota, IMAX)
        r = jnp.min(sel, axis=0, keepdims=True)
        scr[...] = jnp.where(riota == r, IMAX, kk)
        onrow = sub16 == i
        bk = jnp.where(onrow, jnp.broadcast_to(m, (K, QB)), bk)
        bi = jnp.where(onrow, jnp.broadcast_to(r + base, (K, QB)), bi)
        return bk, bi

    riota = jax.lax.broadcasted_iota(jnp.int32, (CB, QB), 0)
    scrs = [kscra, kscrb, kscrc, kscrd]
    for c0 in range(0, NCHUNK, ILP):
        for j in range(ILP):
            scrs[j][...] = keys_for(c0 + j)

        def it(i, carry):
            out = []
            for j in range(ILP):
                bk, bi = step(scrs[j][...], scrs[j], riota, i,
                              carry[2 * j], carry[2 * j + 1],
                              jnp.int32((c0 + j) * CB))
                out += [bk, bi]
            return tuple(out)

        init = []
        for j in range(ILP):
            init += [jnp.full((K, QB), IMAX, jnp.int32), jnp.zeros((K, QB), jnp.int32)]
        res = jax.lax.fori_loop(0, K, it, tuple(init))
        for j in range(ILP):
            candk[pl.ds((c0 + j) * K, K), :] = res[2 * j]
            candi[pl.ds((c0 + j) * K, K), :] = res[2 * j + 1]

    riota_m = jax.lax.broadcasted_iota(jnp.int32, (NCHUNK * K, QB), 0)

    def itm(i, carry):
        bk, bi = carry
        kk = candk[...]
        m = jnp.min(kk, axis=0, keepdims=True)
        sel = jnp.where(kk == m, riota_m, IMAX)
        r = jnp.min(sel, axis=0, keepdims=True)
        gi = jnp.min(jnp.where(riota_m == r, candi[...], IMAX), axis=0, keepdims=True)
        candk[...] = jnp.where(riota_m == r, IMAX, kk)
        onrow = sub16 == i
        bk = jnp.where(onrow, jnp.broadcast_to(m, (K, QB)), bk)
        bi = jnp.where(onrow, jnp.broadcast_to(gi, (K, QB)), bi)
        return bk, bi

    bk, bi = jax.lax.fori_loop(
        0, K, itm, (jnp.full((K, QB), IMAX, jnp.int32), jnp.zeros((K, QB), jnp.int32)))
    dist_ref[...] = jnp.sqrt(jax.lax.bitcast_convert_type(bk, jnp.float32))
    idx_ref[...] = bi


def _knn(p):
    sq = jnp.sum(p * p, axis=1)
    zeros = jnp.zeros((N,), jnp.float32)
    a = jnp.stack([p[:, 0], p[:, 1], p[:, 2], sq] + [zeros] * 12, axis=1)
    a_pad = jnp.zeros((NP - N, 16), jnp.float32).at[:, 3].set(HUGE)
    a = jnp.concatenate([a, a_pad], axis=0)
    b = jnp.stack([p[:, 0], p[:, 1], p[:, 2]] + [zeros] * 13, axis=0)
    b = jnp.concatenate([b, jnp.zeros((16, NP - N), jnp.float32)], axis=1)
    b = b.astype(jnp.bfloat16)
    sqq = jnp.concatenate([sq, jnp.zeros((NP - N,), jnp.float32)])
    sqq = jnp.broadcast_to(sqq[None, :], (8, NP))

    dist16, idx16 = pl.pallas_call(
        _knn_kernel,
        grid=(NBLK,),
        in_specs=[
            pl.BlockSpec((NP, 16), lambda i: (0, 0)),
            pl.BlockSpec((16, QB), lambda i: (0, i)),
            pl.BlockSpec((8, QB), lambda i: (0, i)),
        ],
        out_specs=[
            pl.BlockSpec((K, QB), lambda i: (0, i)),
            pl.BlockSpec((K, QB), lambda i: (0, i)),
        ],
        out_shape=[
            jax.ShapeDtypeStruct((K, NP), jnp.float32),
            jax.ShapeDtypeStruct((K, NP), jnp.int32),
        ],
        scratch_shapes=[
            pltpu.VMEM((CB, QB), jnp.int32),
            pltpu.VMEM((CB, QB), jnp.int32),
            pltpu.VMEM((CB, QB), jnp.int32),
            pltpu.VMEM((CB, QB), jnp.int32),
            pltpu.VMEM((NCHUNK * K, QB), jnp.int32),
            pltpu.VMEM((NCHUNK * K, QB), jnp.int32),
        ],
    )(a, b, sqq)
    return dist16, idx16


def _sc_gather(table, idx_e):
    mesh = plsc.VectorSubcoreMesh(core_axis_name="core", subcore_axis_name="subcore")

    @pl.kernel(out_type=jax.ShapeDtypeStruct((NPK, 256), jnp.float32),
               mesh=mesh)
    def gk(t_hbm, i_hbm, o_hbm):
        def body(i_vmem, o_vmem):
            pltpu.sync_copy(t_hbm.at[i_vmem.at[0]], o_vmem)

        pltpu.emit_pipeline(
            body,
            grid=(NPK // 128,),
            in_specs=[pl.BlockSpec((1, 128), lambda i: (0, i))],
            out_specs=[pl.BlockSpec((128, 256), lambda i: (i, 0))],
            core_axis_name=("core", "subcore"),
            dimension_semantics=(pltpu.PARALLEL,),
        )(i_hbm, o_hbm)

    return gk(table, idx_e)


def _rep16(a):
    return jnp.broadcast_to(a[:, None, :], (QB, K, a.shape[1])).reshape(EB, a.shape[1])


def _valid_mask(i):
    row = jax.lax.broadcasted_iota(jnp.int32, (EB, 1), 0) + i * EB
    return row < NE


def _accum_stats(i, stat_ref, s1, s2):
    pad = jnp.zeros((6, 128), jnp.float32)
    upd = jnp.concatenate([s1, s2, pad], axis=0)

    @pl.when(i == 0)
    def _():
        stat_ref[...] = jnp.zeros((8, 128), jnp.float32)

    stat_ref[...] += upd


def _pad128(v):
    return jnp.concatenate([v, jnp.zeros((1, 128 - v.shape[1]), jnp.float32)], axis=1)


def _p1_kernel(gg_ref, gc_ref, we1_ref, be1_ref, h_ref, stat_ref):
    i = pl.program_id(0)
    gg = gg_ref[...]
    gc = _rep16(gc_ref[...])
    pe = gg[:, 0:3] - gc[:, 0:3]
    dr = jnp.abs(pe[:, 0:1])
    dtheta = jnp.abs(jnp.remainder(pe[:, 1:2] + jnp.pi, 2.0 * jnp.pi) - jnp.pi)
    dz = jnp.abs(pe[:, 2:3])
    ne = gg[:, 3:6] - gc[:, 3:6]
    dn = jnp.sqrt(jnp.sum(ne * ne, axis=1, keepdims=True))
    ef = jnp.concatenate([dn, dtheta, dz, dr] + [jnp.zeros((EB, 12), jnp.float32)],
                         axis=1)
    h = _bf16_dot(ef, we1_ref[...]) + be1_ref[...]
    h_ref[...] = h
    hm = jnp.where(_valid_mask(i), h, 0.0)
    s1 = jnp.sum(hm, axis=0, keepdims=True)
    s2 = jnp.sum(hm * hm, axis=0, keepdims=True)
    _accum_stats(i, stat_ref, _pad128(s1), _pad128(s2))


def _p2_kernel(xg_ref, h_ref, expd_ref, xc_ref, bn1_ref,
               wq_ref, bq_ref, wk_ref, bk_ref, wv_ref, bv_ref,
               we2_ref, be2_ref, wpre_ref, av_ref, stat_ref):
    i = pl.program_id(0)
    xc = xc_ref[...]
    xq_c = _bf16_dot(xc, wq_ref[...]) + bq_ref[...]
    xe = xg_ref[...].astype(jnp.float32) - _rep16(xc)
    xk = _bf16_dot(xe, wk_ref[...]) + bk_ref[...]
    xv = _bf16_dot(xe, wv_ref[...]) + bv_ref[...]
    h = h_ref[...] * bn1_ref[0:1, 0:16] + bn1_ref[1:2, 0:16]
    h = jnp.maximum(h, 0.0)
    emb = _bf16_dot(h, we2_ref[...]) + be2_ref[...]
    wpre = (_rep16(xq_c) - xk) + emb
    wpre_ref[...] = wpre
    expd = jnp.broadcast_to(expd_ref[...], (EB, C_OUT))
    av_ref[...] = xv * expd + emb
    wm = jnp.where(_valid_mask(i), wpre, 0.0)
    s1 = jnp.sum(wm, axis=0, keepdims=True)
    s2 = jnp.sum(wm * wm, axis=0, keepdims=True)
    _accum_stats(i, stat_ref, s1, s2)


def _p3_kernel(wpre_ref, bn2_ref, ww1_ref, bw1_ref, w1_ref, stat_ref):
    i = pl.program_id(0)
    wb = wpre_ref[...] * bn2_ref[0:1, :] + bn2_ref[1:2, :]
    wb = jnp.maximum(wb, 0.0)
    w1 = _bf16_dot(wb, ww1_ref[...]) + bw1_ref[...]
    w1_ref[...] = w1
    wm = jnp.where(_valid_mask(i), w1, 0.0)
    s1 = jnp.sum(wm, axis=0, keepdims=True)
    s2 = jnp.sum(wm * wm, axis=0, keepdims=True)
    _accum_stats(i, stat_ref, _pad128(s1), _pad128(s2))


def _p4_kernel(w1_ref, av_ref, bn3_ref, ww2_ref, bw2_ref, out_ref):
    w1 = w1_ref[...] * bn3_ref[0:1, 0:16] + bn3_ref[1:2, 0:16]
    w1 = jnp.maximum(w1, 0.0)
    w2 = _bf16_dot(w1, ww2_ref[...]) + bw2_ref[...]
    w3 = w2.reshape(QB, K, HID)
    m = jnp.max(w3, axis=1, keepdims=True)
    e = jnp.exp(w3 - m)
    w = e / jnp.sum(e, axis=1, keepdims=True)
    wt = jnp.concatenate([w] * S, axis=2)
    av = av_ref[...].reshape(QB, K, C_OUT)
    out_ref[...] = jnp.sum(av * wt, axis=1)


def _bn_params(stat, denom, g, b, nch):
    s1 = stat[0, :nch]
    s2 = stat[1, :nch]
    mean = s1 / denom
    var = s2 / denom - mean * mean
    scale = g / jnp.sqrt(var + EPS)
    shift = b - mean * scale
    lanes = jnp.zeros((2, 128), jnp.float32)
    lanes = lanes.at[0, :nch].set(scale).at[1, :nch].set(shift)
    return jnp.concatenate([lanes, jnp.zeros((6, 128), jnp.float32)], axis=0)


def _edge_grid_call(kernel_fn, in_arrs, in_specs, out_specs, out_shape):
    return pl.pallas_call(
        kernel_fn,
        grid=(NBLK,),
        in_specs=in_specs,
        out_specs=out_specs,
        out_shape=out_shape,
    )(*in_arrs)


def kernel(p, n, x, o, Wq, bq, Wk, bk, Wv, bv, We1, be1, g_e, b_e, We2, be2, g_w0, b_w0, Ww1, bw1, g_w1, b_w1, Ww2, bw2):
    f32 = jnp.float32
    dist16, idx16 = _knn(p)
    idx = idx16[:, :N].T
    dist = dist16[:, :N].T

    idx_e = jnp.concatenate([idx.reshape(-1), jnp.zeros((NPK - NE,), jnp.int32)])
    idx_e = idx_e[None, :]
    expd = jnp.exp(-dist).reshape(-1)
    expd_e = jnp.concatenate([expd, jnp.zeros((NPK - NE,), f32)])[:, None]

    x_pad = jnp.concatenate([x, jnp.zeros((NP - N, C_IN), f32)], axis=0)
    r = jnp.sqrt(p[:, 0] ** 2 + p[:, 1] ** 2)
    theta = jnp.arctan2(p[:, 1], p[:, 0])
    geo6 = jnp.stack([r, theta, p[:, 2], n[:, 0], n[:, 1], n[:, 2]], axis=1)
    geo6 = jnp.concatenate([geo6, jnp.zeros((NP - N, 6), f32)], axis=0)
    geo = jnp.concatenate([geo6, jnp.zeros((NP, 10), f32)], axis=1)
    table = jnp.concatenate([x_pad, geo6, jnp.zeros((NP, 122), f32)], axis=1)

    tg = _sc_gather(table, idx_e)

    full = lambda shp: pl.BlockSpec(shp, lambda i: tuple(0 for _ in shp))
    eb_blk = lambda L: pl.BlockSpec((EB, L), lambda i: (i, 0))
    qb_blk = lambda L: pl.BlockSpec((QB, L), lambda i: (i, 0))
    acc_spec = pl.BlockSpec((8, 128), lambda i: (0, 0))

    We1_16 = jnp.concatenate([We1, jnp.zeros((12, 16), f32)], axis=0)
    h_pre, st1 = _edge_grid_call(
        _p1_kernel,
        [tg, geo, We1_16, be1[None, :]],
        [pl.BlockSpec((EB, 128), lambda i: (i, 1)), qb_blk(16), full((16, 16)), full((1, 16))],
        [pl.BlockSpec((EB, 16), lambda i: (i, 0)), acc_spec],
        [jax.ShapeDtypeStruct((NPK, 16), f32), jax.ShapeDtypeStruct((8, 128), f32)],
    )
    bn1 = _bn_params(st1, float(NE), g_e, b_e, 16)

    w_pre, a_v, st2 = _edge_grid_call(
        _p2_kernel,
        [tg, h_pre, expd_e, x_pad, bn1,
         Wq, bq[None, :], Wk, bk[None, :], Wv, bv[None, :], We2, be2[None, :]],
        [pl.BlockSpec((EB, C_IN), lambda i: (i, 0)), pl.BlockSpec((EB, 16), lambda i: (i, 0)),
         pl.BlockSpec((EB, 1), lambda i: (i, 0)), qb_blk(C_IN), full((8, 128)),
         full((C_IN, MID)), full((1, MID)), full((C_IN, MID)), full((1, MID)),
         full((C_IN, C_OUT)), full((1, C_OUT)), full((16, C_OUT)), full((1, C_OUT))],
        [eb_blk(MID), eb_blk(C_OUT), acc_spec],
        [jax.ShapeDtypeStruct((NPK, MID), f32), jax.ShapeDtypeStruct((NPK, C_OUT), f32),
         jax.ShapeDtypeStruct((8, 128), f32)],
    )
    bn2 = _bn_params(st2, float(NE), g_w0, b_w0, 128)

    w1_pre, st3 = _edge_grid_call(
        _p3_kernel,
        [w_pre, bn2, Ww1, bw1[None, :]],
        [eb_blk(MID), full((8, 128)), full((MID, HID)), full((1, HID))],
        [pl.BlockSpec((EB, HID), lambda i: (i, 0)), acc_spec],
        [jax.ShapeDtypeStruct((NPK, HID), f32), jax.ShapeDtypeStruct((8, 128), f32)],
    )
    bn3 = _bn_params(st3, float(NE), g_w1, b_w1, HID)

    out = _edge_grid_call(
        _p4_kernel,
        [w1_pre, a_v, bn3, Ww2, bw2[None, :]],
        [pl.BlockSpec((EB, HID), lambda i: (i, 0)), eb_blk(C_OUT), full((8, 128)),
         full((HID, HID)), full((1, HID))],
        qb_blk(C_OUT),
        jax.ShapeDtypeStruct((NP, C_OUT), f32),
    )
    return out[:N]
